# baseline (speedup 1.0000x reference)
import functools

import jax
import jax.numpy as jnp
from jax import lax
from jax.experimental import pallas as pl
from jax.experimental.pallas import tpu as pltpu
from jax.experimental.pallas import tpu_sc as plsc

NBUF = 2


@functools.cache
def _make_gather(V2, NJ, NI):
    info = plsc.get_sparse_core_info()
    NC, NS, L = info.num_cores, info.num_subcores, info.num_lanes
    NW = NC * NS
    assert NI % NW == 0
    W = NI // NW
    assert W % L == 0
    D = 64
    mesh = plsc.VectorSubcoreMesh(core_axis_name="c", subcore_axis_name="s")

    @functools.partial(
        pl.kernel,
        mesh=mesh,
        out_type=jax.ShapeDtypeStruct((NJ, D, NI), jnp.float32),
        scratch_types=[
            [pltpu.VMEM((W,), jnp.int32)] * NBUF,
            [pltpu.VMEM((W,), jnp.int32)] * NBUF,
            [pltpu.VMEM((W,), jnp.int32)] * NBUF,
            [pltpu.VMEM((W, 2 * D), jnp.float32)] * NBUF,
            [pltpu.VMEM((D, W + 1), jnp.float32)] * NBUF,
            [pltpu.SemaphoreType.DMA] * NBUF,
            [pltpu.SemaphoreType.DMA] * NBUF,
            [pltpu.SemaphoreType.DMA] * NBUF,
        ],
        compiler_params=pltpu.CompilerParams(
            use_tc_tiling_on_sc=True, needs_layout_passes=False
        ),
    )
    def gather_kernel(t2_hbm, idxt_hbm, out_hbm,
                      idxraw, idxrow, half, rows, panel, isems, gsems, wsems):
        wid = lax.axis_index("s") * NC + lax.axis_index("c")
        i0 = wid * W

        def start_idx(j, b):
            pltpu.async_copy(idxt_hbm.at[j, pl.ds(i0, W)], idxraw[b], isems[b])

        def wait_idx(j, b):
            pltpu.make_async_copy(
                idxt_hbm.at[j, pl.ds(i0, W)], idxraw[b], isems[b]
            ).wait()

        def prep_and_gather(b):
            for g in range(W // L):
                ids = idxraw[b][pl.ds(g * L, L)]
                idxrow[b][pl.ds(g * L, L)] = lax.shift_right_logical(ids, 1)
                half[b][pl.ds(g * L, L)] = lax.shift_left(
                    lax.bitwise_and(ids, 1), 6
                )
            pltpu.async_copy(t2_hbm.at[idxrow[b]], rows[b], gsems[b])

        def wait_gather(b):
            pltpu.make_async_copy(t2_hbm.at[idxrow[b]], rows[b], gsems[b]).wait()

        def transpose(b):
            dio = lax.iota(jnp.int32, L)
            diod = [dio + d0 for d0 in range(0, D, L)]
            zero = jnp.full((L,), 0, jnp.int32)

            @plsc.parallel_loop(0, W // L)
            def gbody(g):
                cb_vec = half[b][pl.ds(g * L, L)]
                for lp in range(L):
                    row = g * L + lp
                    cb = cb_vec[lp]
                    rvec = zero + row
                    for k in range(D // L):
                        vals = rows[b][row, pl.ds(cb + k * L, L)]
                        plsc.store_scatter(panel[b], [diod[k], rvec], vals)

        def start_write(j, b):
            pltpu.async_copy(
                panel[b].at[:, pl.ds(0, W)], out_hbm.at[j, :, pl.ds(i0, W)],
                wsems[b],
            )

        def wait_write(j, b):
            pltpu.make_async_copy(
                panel[b].at[:, pl.ds(0, W)], out_hbm.at[j, :, pl.ds(i0, W)],
                wsems[b],
            ).wait()

        for b in range(NBUF):
            start_idx(b, b)
        for b in range(NBUF):
            wait_idx(b, b)
            prep_and_gather(b)

        def body(it, carry):
            for b in range(NBUF):
                j = it * NBUF + b

                @pl.when(j + NBUF < NJ)
                def _():
                    start_idx(j + NBUF, b)

                wait_gather(b)

                @pl.when(j >= NBUF)
                def _():
                    wait_write(j - NBUF, b)

                transpose(b)
                start_write(j, b)

                @pl.when(j + NBUF < NJ)
                def _():
                    wait_idx(j + NBUF, b)
                    prep_and_gather(b)

            return carry

        lax.fori_loop(0, NJ // NBUF, body, 0)

        for b in range(NBUF):
            wait_write(NJ - NBUF + b, b)

    return gather_kernel


def kernel(token_ids, table):
    V, D = table.shape
    NI, NJ = token_ids.shape
    t2 = table.reshape(V // 2, 2 * D)
    idxt = jnp.swapaxes(token_ids, 0, 1).astype(jnp.int32)
    out = _make_gather(V // 2, NJ, NI)(t2, idxt)
    return jnp.transpose(out, (2, 0, 1))


# speedup vs baseline: 1.1444x; 1.1444x over previous
import functools

import jax
import jax.numpy as jnp
from jax import lax
from jax.experimental import pallas as pl
from jax.experimental.pallas import tpu as pltpu
from jax.experimental.pallas import tpu_sc as plsc


@functools.cache
def _make_gather(V, D, B, C, NBUF):
    info = plsc.get_sparse_core_info()
    NC, NS = info.num_cores, info.num_subcores
    NW = NC * NS
    assert B % NW == 0
    b_per_w = B // NW
    assert b_per_w % C == 0
    n_chunks = b_per_w // C
    assert n_chunks % NBUF == 0 and n_chunks >= 2 * NBUF
    mesh = plsc.VectorSubcoreMesh(core_axis_name="c", subcore_axis_name="s")

    @functools.partial(
        pl.kernel,
        mesh=mesh,
        out_type=jax.ShapeDtypeStruct((B, D), jnp.float32),
        scratch_types=[
            pltpu.VMEM((b_per_w,), jnp.int32),
            pltpu.VMEM((NBUF, C, D), jnp.float32),
            [pltpu.SemaphoreType.DMA] * NBUF,
            [pltpu.SemaphoreType.DMA] * NBUF,
        ],
        compiler_params=pltpu.CompilerParams(use_tc_tiling_on_sc=False),
    )
    def gather_kernel(table_hbm, idx_hbm, out_hbm, idx_v, rows_v, gsems, wsems):
        wid = lax.axis_index("s") * NC + lax.axis_index("c")
        base = wid * b_per_w
        pltpu.sync_copy(idx_hbm.at[pl.ds(base, b_per_w)], idx_v)

        def start_gather(c, b):
            pltpu.async_copy(
                table_hbm.at[idx_v.at[pl.ds(c * C, C)]], rows_v.at[b], gsems[b]
            )

        def wait_gather(c, b):
            pltpu.make_async_copy(
                table_hbm.at[idx_v.at[pl.ds(c * C, C)]], rows_v.at[b], gsems[b]
            ).wait()

        def start_write(c, b):
            pltpu.async_copy(
                rows_v.at[b], out_hbm.at[pl.ds(base + c * C, C)], wsems[b]
            )

        def wait_write(c, b):
            pltpu.make_async_copy(
                rows_v.at[b], out_hbm.at[pl.ds(base + c * C, C)], wsems[b]
            ).wait()

        def body(g, carry):
            for b in range(NBUF):
                c = g * NBUF + b
                @pl.when(c >= NBUF)
                def _():
                    wait_write(c - NBUF, b)

                start_gather(c, b)

                cw = c - (NBUF - 1)
                bw = (b - (NBUF - 1)) % NBUF

                @pl.when(cw >= 0)
                def _():
                    wait_gather(cw, bw)
                    start_write(cw, bw)

            return carry

        lax.fori_loop(0, n_chunks // NBUF, body, 0)

        for j in range(NBUF - 1):
            cw = n_chunks - (NBUF - 1) + j
            bw = cw % NBUF
            wait_gather(cw, bw)
            start_write(cw, bw)
        for j in range(NBUF):
            c = n_chunks - NBUF + j
            wait_write(c, c % NBUF)

    return gather_kernel


def kernel(token_ids, table):
    V, D = table.shape
    B = token_ids.size
    idx = token_ids.reshape(B).astype(jnp.int32)
    out = _make_gather(V, D, B, 256, 4)(table, idx)
    return out.reshape(*token_ids.shape, D)
